# baseline (speedup 1.0000x reference)
import functools

import jax
import jax.numpy as jnp
from jax import lax
from jax.experimental import pallas as pl
from jax.experimental.pallas import tpu as pltpu
from jax.experimental.pallas import tpu_sc as plsc

BATCH = 4096
SEQ = 200
DIM = 64
NW = 32
BW = BATCH // NW
L = 16


def _build():
    mesh = plsc.VectorSubcoreMesh(core_axis_name="c", subcore_axis_name="s")

    @functools.partial(
        pl.kernel,
        mesh=mesh,
        out_type=jax.ShapeDtypeStruct((SEQ, DIM, BATCH), jnp.float32),
        scratch_types=[
            pltpu.VMEM((SEQ, BW), jnp.int32),
            pltpu.VMEM((SEQ, DIM), jnp.float32),
            pltpu.VMEM((BW,), jnp.int32),
            pltpu.VMEM((BW, 2 * DIM), jnp.float32),
            pltpu.VMEM((DIM, BW), jnp.float32),
            pltpu.SemaphoreType.DMA,
        ],
        compiler_params=pltpu.CompilerParams(
            use_tc_tiling_on_sc=True, needs_layout_passes=False),
    )
    def emb(ids_hbm, table_hbm, pos_hbm, out_hbm, ids_v, pos_v, idx2_v,
            pair_v, obuf, sem):
        wid = lax.axis_index("s") * 2 + lax.axis_index("c")
        b0 = wid * BW
        pltpu.sync_copy(pos_hbm, pos_v)
        pltpu.sync_copy(ids_hbm.at[:, pl.ds(b0, BW)], ids_v)

        lanes = lax.iota(jnp.int32, L)
        rows = [lanes + L * g for g in range(BW // L)]

        def task(s, carry):
            cbase = []
            for g in range(BW // L):
                t = ids_v[s, pl.ds(L * g, L)]
                idx2_v[pl.ds(L * g, L)] = lax.shift_right_logical(t, 1)
                cbase.append(lax.shift_left(jnp.bitwise_and(t, 1), 6))
            pltpu.async_copy(table_hbm.at[idx2_v], pair_v, sem).wait()

            def col_body(d, c):
                dsplat = jnp.full((L,), d, jnp.int32)
                psplat = plsc.load_gather(
                    pos_v, (jnp.full((L,), s, jnp.int32), dsplat))
                for g in range(BW // L):
                    v = plsc.load_gather(pair_v, (rows[g], cbase[g] + dsplat))
                    obuf[d, pl.ds(L * g, L)] = v + psplat
                return c

            lax.fori_loop(0, DIM, col_body, 0)
            pltpu.sync_copy(obuf, out_hbm.at[s, :, pl.ds(b0, BW)])
            return carry

        lax.fori_loop(0, SEQ, task, 0)

    return emb


_emb = _build()


def kernel(token_ids, token_table, pos_table):
    ids_t = token_ids.astype(jnp.int32).T
    table_p = token_table.reshape(VOCAB_PAIRS, 2 * DIM)
    out = _emb(ids_t, table_p, pos_table)
    return out.transpose(2, 0, 1)


VOCAB_PAIRS = 500000


# speedup vs baseline: 1.6810x; 1.6810x over previous
import functools

import jax
import jax.numpy as jnp
from jax import lax
from jax.experimental import pallas as pl
from jax.experimental.pallas import tpu as pltpu
from jax.experimental.pallas import tpu_sc as plsc

BATCH = 4096
SEQ = 200
DIM = 64
NW = 32
BW = BATCH // NW
L = 16
NG = BW // L
VOCAB_PAIRS = 500000


def _build():
    mesh = plsc.VectorSubcoreMesh(core_axis_name="c", subcore_axis_name="s")

    @functools.partial(
        pl.kernel,
        mesh=mesh,
        out_type=jax.ShapeDtypeStruct((SEQ, DIM, BATCH), jnp.float32),
        scratch_types=[
            pltpu.VMEM((SEQ, BW), jnp.int32),
            pltpu.VMEM((SEQ, DIM), jnp.float32),
            [pltpu.VMEM((BW,), jnp.int32) for _ in range(2)],
            [pltpu.VMEM((BW, 2 * DIM), jnp.float32) for _ in range(2)],
            [pltpu.VMEM((DIM, BW), jnp.float32) for _ in range(2)],
            [pltpu.SemaphoreType.DMA for _ in range(2)],
            [pltpu.SemaphoreType.DMA for _ in range(2)],
        ],
        compiler_params=pltpu.CompilerParams(
            use_tc_tiling_on_sc=True, needs_layout_passes=False),
    )
    def emb(ids_hbm, table_hbm, pos_hbm, out_hbm, ids_v, pos_v, idx2_v,
            pair_v, obuf, gsem, wsem):
        wid = lax.axis_index("s") * 2 + lax.axis_index("c")
        b0 = wid * BW
        pltpu.sync_copy(pos_hbm, pos_v)
        pltpu.sync_copy(ids_hbm.at[:, pl.ds(b0, BW)], ids_v)

        lanes = lax.iota(jnp.int32, L)
        rows = [lanes + L * g for g in range(NG)]

        def prep_and_fire(s, slot):
            for g in range(NG):
                t = ids_v[s, pl.ds(L * g, L)]
                idx2_v[slot][pl.ds(L * g, L)] = lax.shift_right_logical(t, 1)
            pltpu.async_copy(table_hbm.at[idx2_v[slot]], pair_v[slot],
                             gsem[slot])

        def wait_wb(slot):
            pltpu.make_async_copy(
                obuf[slot], out_hbm.at[0, :, pl.ds(b0, BW)], wsem[slot]
            ).wait()

        prep_and_fire(0, 0)

        def iter_body(i, carry):
            for q in range(2):
                s = 2 * i + q
                sn = s + 1
                slot_n = 1 - q

                if q == 0:
                    @pl.when(jnp.logical_and(sn < SEQ, i > 0))
                    def _():
                        wait_wb(slot_n)
                        prep_and_fire(sn, slot_n)

                    @pl.when(jnp.logical_and(sn < SEQ, i == 0))
                    def _():
                        prep_and_fire(sn, slot_n)

                else:
                    @pl.when(sn < SEQ)
                    def _():
                        wait_wb(slot_n)
                        prep_and_fire(sn, slot_n)

                pltpu.make_async_copy(
                    table_hbm.at[idx2_v[q]], pair_v[q], gsem[q]).wait()

                cbase = []
                for g in range(NG):
                    t = ids_v[s, pl.ds(L * g, L)]
                    cbase.append(lax.shift_left(jnp.bitwise_and(t, 1), 6))
                pv = pair_v[q]
                ob = obuf[q]
                ssplat = jnp.full((L,), s, jnp.int32)

                @plsc.parallel_loop(0, DIM, step=1, unroll=4)
                def col_body(d):
                    dsplat = jnp.full((L,), d, jnp.int32)
                    psplat = plsc.load_gather(pos_v, (ssplat, dsplat))
                    for g in range(NG):
                        v = plsc.load_gather(pv, (rows[g], cbase[g] + dsplat))
                        ob[d, pl.ds(L * g, L)] = v + psplat

                pltpu.async_copy(ob, out_hbm.at[s, :, pl.ds(b0, BW)], wsem[q])
            return carry

        lax.fori_loop(0, SEQ // 2, iter_body, 0)
        for q in range(2):
            wait_wb(q)

    return emb


_emb = _build()


def kernel(token_ids, token_table, pos_table):
    ids_t = token_ids.astype(jnp.int32).T
    table_p = token_table.reshape(VOCAB_PAIRS, 2 * DIM)
    out = _emb(ids_t, table_p, pos_table)
    return out.transpose(2, 0, 1)


# speedup vs baseline: 1.7031x; 1.0132x over previous
import functools

import jax
import jax.numpy as jnp
from jax import lax
from jax.experimental import pallas as pl
from jax.experimental.pallas import tpu as pltpu
from jax.experimental.pallas import tpu_sc as plsc

BATCH = 4096
SEQ = 200
DIM = 64
NW = 32
BW = BATCH // NW
L = 16
NG = BW // L
NPB = 4
NOB = 2
AH = 3
OST = BW + 1
VOCAB_PAIRS = 500000


def _build():
    mesh = plsc.VectorSubcoreMesh(core_axis_name="c", subcore_axis_name="s")

    @functools.partial(
        pl.kernel,
        mesh=mesh,
        out_type=jax.ShapeDtypeStruct((SEQ, DIM, BATCH), jnp.float32),
        scratch_types=[
            pltpu.VMEM((SEQ, DIM), jnp.float32),
            pltpu.VMEM((BW,), jnp.int32),
            [pltpu.VMEM((BW,), jnp.int32) for _ in range(NPB)],
            [pltpu.VMEM((BW,), jnp.int32) for _ in range(NPB)],
            [pltpu.VMEM((BW, 2 * DIM), jnp.float32) for _ in range(NPB)],
            [pltpu.VMEM((DIM, OST), jnp.float32) for _ in range(NOB)],
            [pltpu.SemaphoreType.DMA for _ in range(NPB)],
            [pltpu.SemaphoreType.DMA for _ in range(NPB)],
            [pltpu.SemaphoreType.DMA for _ in range(NOB)],
        ],
        compiler_params=pltpu.CompilerParams(
            use_tc_tiling_on_sc=True, needs_layout_passes=False),
    )
    def emb(ids_hbm, table_hbm, pos_hbm, out_hbm, pos_v, par_v, raw_v,
            idx2_v, pair_v, obuf, isem, gsem, wsem):
        wid = lax.axis_index("s") * 2 + lax.axis_index("c")
        b0 = wid * BW
        pltpu.sync_copy(pos_hbm, pos_v)

        lanes = lax.iota(jnp.int32, L)
        ck = [lanes + L * k for k in range(DIM // L)]

        def fire_ids(s, slot):
            pltpu.async_copy(ids_hbm.at[s, pl.ds(b0, BW)], raw_v[slot],
                             isem[slot])

        def wait_ids(slot):
            pltpu.make_async_copy(ids_hbm.at[0, pl.ds(b0, BW)], raw_v[slot],
                                  isem[slot]).wait()

        def fire_gather(s, slot):
            for g in range(NG):
                t = raw_v[slot][pl.ds(L * g, L)]
                idx2_v[slot][pl.ds(L * g, L)] = lax.shift_right_logical(t, 1)
            pltpu.async_copy(table_hbm.at[idx2_v[slot]], pair_v[slot],
                             gsem[slot])

        def wait_gather(slot):
            pltpu.make_async_copy(
                table_hbm.at[idx2_v[slot]], pair_v[slot], gsem[slot]).wait()

        def wait_wb(slot):
            pltpu.make_async_copy(
                obuf[slot].at[:, pl.ds(0, BW)],
                out_hbm.at[0, :, pl.ds(b0, BW)], wsem[slot]
            ).wait()

        for s0 in range(NPB):
            fire_ids(s0, s0)
        for s0 in range(AH):
            wait_ids(s0)
            fire_gather(s0, s0)

        def iter_body(i, carry):
            for q in range(NPB):
                s = NPB * i + q
                wait_gather(q)

                for g in range(NG):
                    t = raw_v[q][pl.ds(L * g, L)]
                    par_v[pl.ds(L * g, L)] = jnp.bitwise_and(t, 1)

                gslot = (q + AH) % NPB

                @pl.when(s + AH < SEQ)
                def _():
                    wait_ids(gslot)
                    fire_gather(s + AH, gslot)

                @pl.when(s + NPB < SEQ)
                def _():
                    fire_ids(s + NPB, q)

                pk = [pos_v[s, pl.ds(L * k, L)] for k in range(DIM // L)]
                pv = pair_v[q]
                oq = q % NOB
                ob = obuf[oq]

                @pl.when(jnp.logical_or(i > 0, q >= NOB))
                def _():
                    wait_wb(oq)

                @plsc.parallel_loop(0, BW, step=1, unroll=2)
                def b_body(b):
                    bsplat = jnp.full((L,), b, jnp.int32)
                    pars = plsc.load_gather(par_v, (bsplat,))
                    c0 = lax.shift_left(pars, 6)
                    for k in range(DIM // L):
                        v = plsc.load_gather(pv, (bsplat, c0 + ck[k]))
                        plsc.store_scatter(ob, (ck[k], bsplat), v + pk[k])

                pltpu.async_copy(ob.at[:, pl.ds(0, BW)],
                                 out_hbm.at[s, :, pl.ds(b0, BW)], wsem[oq])
            return carry

        lax.fori_loop(0, SEQ // NPB, iter_body, 0)
        for oq in range(NOB):
            wait_wb(oq)

    return emb


_emb = _build()


def kernel(token_ids, token_table, pos_table):
    ids_t = token_ids.astype(jnp.int32).T
    table_p = token_table.reshape(VOCAB_PAIRS, 2 * DIM)
    out = _emb(ids_t, table_p, pos_table)
    return out.transpose(2, 0, 1)
